# baseline (speedup 1.0000x reference)
import functools

import jax
import jax.numpy as jnp
from jax import lax
from jax.experimental import pallas as pl
from jax.experimental.pallas import tpu as pltpu
from jax.experimental.pallas import tpu_sc as plsc

NUM_CORES = 2
NUM_SUBCORES = 16
LANES = 16
NW = NUM_CORES * NUM_SUBCORES


def _mf_body(users_hbm, items_hbm, utab_hbm, itab_hbm, out_hbm,
             idx_u, idx_i, urows, irows, outv, sem_u, sem_i):
    b_per_w = idx_u.shape[0]
    wid = lax.axis_index("s") * NUM_CORES + lax.axis_index("c")
    base = wid * b_per_w

    pltpu.sync_copy(users_hbm.at[pl.ds(base, b_per_w)], idx_u)
    pltpu.sync_copy(items_hbm.at[pl.ds(base, b_per_w)], idx_i)

    cp_u = pltpu.async_copy(utab_hbm.at[idx_u], urows, sem_u)
    cp_i = pltpu.async_copy(itab_hbm.at[idx_i], irows, sem_i)
    cp_u.wait()
    cp_i.wait()

    nblk = b_per_w // LANES

    @pl.loop(0, nblk)
    def _blocks(blk):
        rows = blk * LANES + lax.iota(jnp.int32, LANES)
        acc = jnp.zeros((LANES,), jnp.float32)
        for d in range(LANES):
            cols = jnp.full((LANES,), d, jnp.int32)
            uu = plsc.load_gather(urows, [rows, cols])
            vv = plsc.load_gather(irows, [rows, cols])
            acc = acc + uu * vv
        outv[pl.ds(blk * LANES, LANES)] = 1.0 / (1.0 + jnp.exp(-acc))

    pltpu.sync_copy(outv, out_hbm.at[pl.ds(base, b_per_w)])


def kernel(users, items, user_table, item_table):
    batch = users.shape[0]
    dim = user_table.shape[1]
    b_per_w = batch // NW
    mesh = plsc.VectorSubcoreMesh(
        core_axis_name="c", subcore_axis_name="s",
        num_cores=NUM_CORES, num_subcores=NUM_SUBCORES)
    run = functools.partial(
        pl.kernel,
        out_type=jax.ShapeDtypeStruct((batch,), jnp.float32),
        mesh=mesh,
        compiler_params=pltpu.CompilerParams(
            needs_layout_passes=False, use_tc_tiling_on_sc=False),
        scratch_types=[
            pltpu.VMEM((b_per_w,), jnp.int32),
            pltpu.VMEM((b_per_w,), jnp.int32),
            pltpu.VMEM((b_per_w, dim), jnp.float32),
            pltpu.VMEM((b_per_w, dim), jnp.float32),
            pltpu.VMEM((b_per_w,), jnp.float32),
            pltpu.SemaphoreType.DMA,
            pltpu.SemaphoreType.DMA,
        ],
    )(_mf_body)
    return run(users, items, user_table, item_table)
